# baseline (speedup 1.0000x reference)
import functools

import jax
import jax.numpy as jnp
from jax.experimental import pallas as pl
from jax.experimental.pallas import tpu as pltpu

N = 4096
D = 128
IN_DIM = 384
BM = 512
BK = 512
NI = N // BM
NK = N // BK
EPS = 1e-12


def _dot(a, b):
    return jax.lax.dot_general(
        a.astype(jnp.bfloat16), b.astype(jnp.bfloat16),
        (((1,), (0,)), ((), ())), preferred_element_type=jnp.float32,
    )


def _proj_kernel(uf, bf, wu, wb, u0, b0):
    u0[...] = _dot(uf[...], wu[...])
    b0[...] = _dot(bf[...], wb[...])


def _prop1_kernel(abu, auu, aub, abb, u, b, u0b, b0b, u1, b1, su, sb):
    k = pl.program_id(1)

    @pl.when(k == 0)
    def _():
        u1[...] = jnp.zeros_like(u1)
        b1[...] = jnp.zeros_like(b1)

    uk = u[pl.ds(k * BK, BK), :]
    bk = b[pl.ds(k * BK, BK), :]
    u1[...] += _dot(abu[...], bk) + _dot(auu[...], uk)
    b1[...] += _dot(aub[...], uk) + _dot(abb[...], bk)

    @pl.when(k == NK - 1)
    def _():
        su[...] = u0b[...] + u1[...]
        sb[...] = b0b[...] + b1[...]


def _prop2_kernel(abu, auu, aub, abb, u, b, sub, sbb, uh, bh, accu, accb):
    k = pl.program_id(1)

    @pl.when(k == 0)
    def _():
        accu[...] = jnp.zeros_like(accu)
        accb[...] = jnp.zeros_like(accb)

    uk = u[pl.ds(k * BK, BK), :]
    bk = b[pl.ds(k * BK, BK), :]
    accu[...] += _dot(abu[...], bk) + _dot(auu[...], uk)
    accb[...] += _dot(aub[...], uk) + _dot(abb[...], bk)

    @pl.when(k == NK - 1)
    def _():
        emb_u = (sub[...] + accu[...]) * (1.0 / 3.0)
        emb_b = (sbb[...] + accb[...]) * (1.0 / 3.0)
        nu = jnp.sqrt(jnp.sum(emb_u * emb_u, axis=-1, keepdims=True))
        nb = jnp.sqrt(jnp.sum(emb_b * emb_b, axis=-1, keepdims=True))
        uh[...] = emb_u / jnp.maximum(nu, EPS)
        bh[...] = emb_b / jnp.maximum(nb, EPS)


def _adj_spec():
    return pl.BlockSpec((BM, BK), lambda i, k: (i, k))


def _resident_spec():
    return pl.BlockSpec((N, D), lambda i, k: (0, 0))


def _row_spec():
    return pl.BlockSpec((BM, D), lambda i, k: (i, 0))


@functools.partial(jax.jit)
def kernel(user_feat, biz_feat, adj_ub, adj_bu, adj_uu, adj_bb, W_user, W_biz):
    emb = jax.ShapeDtypeStruct((N, D), jnp.float32)
    blk = jax.ShapeDtypeStruct((BM, D), jnp.float32)

    u0, b0 = pl.pallas_call(
        _proj_kernel,
        grid=(NI,),
        in_specs=[
            pl.BlockSpec((BM, IN_DIM), lambda i: (i, 0)),
            pl.BlockSpec((BM, IN_DIM), lambda i: (i, 0)),
            pl.BlockSpec((IN_DIM, D), lambda i: (0, 0)),
            pl.BlockSpec((IN_DIM, D), lambda i: (0, 0)),
        ],
        out_specs=[
            pl.BlockSpec((BM, D), lambda i: (i, 0)),
            pl.BlockSpec((BM, D), lambda i: (i, 0)),
        ],
        out_shape=[emb, emb],
        compiler_params=pltpu.CompilerParams(
            dimension_semantics=("parallel",),
        ),
    )(user_feat, biz_feat, W_user, W_biz)

    u1, b1, su, sb = pl.pallas_call(
        _prop1_kernel,
        grid=(NI, NK),
        in_specs=[
            _adj_spec(), _adj_spec(), _adj_spec(), _adj_spec(),
            _resident_spec(), _resident_spec(),
            _row_spec(), _row_spec(),
        ],
        out_specs=[_row_spec(), _row_spec(), _row_spec(), _row_spec()],
        out_shape=[emb, emb, emb, emb],
        compiler_params=pltpu.CompilerParams(
            dimension_semantics=("parallel", "arbitrary"),
        ),
    )(adj_bu, adj_uu, adj_ub, adj_bb, u0, b0, u0, b0)

    user_h, biz_h = pl.pallas_call(
        _prop2_kernel,
        grid=(NI, NK),
        in_specs=[
            _adj_spec(), _adj_spec(), _adj_spec(), _adj_spec(),
            _resident_spec(), _resident_spec(),
            _row_spec(), _row_spec(),
        ],
        out_specs=[_row_spec(), _row_spec()],
        out_shape=[emb, emb],
        scratch_shapes=[
            pltpu.VMEM((BM, D), jnp.float32),
            pltpu.VMEM((BM, D), jnp.float32),
        ],
        compiler_params=pltpu.CompilerParams(
            dimension_semantics=("parallel", "arbitrary"),
        ),
    )(adj_bu, adj_uu, adj_ub, adj_bb, u1, b1, su, sb)

    return (user_h, biz_h)


# speedup vs baseline: 1.1789x; 1.1789x over previous
import jax
import jax.numpy as jnp
from jax.experimental import pallas as pl
from jax.experimental.pallas import tpu as pltpu

N = 4096
D = 128
IN_DIM = 384
BM = 512
NB = N // BM
EPS = 1e-12


def _dot(a, b):
    return jax.lax.dot_general(
        a.astype(jnp.bfloat16), b.astype(jnp.bfloat16),
        (((1,), (0,)), ((), ())), preferred_element_type=jnp.float32,
    )


def _proj_kernel(uf, bf, wu, wb, u0, b0):
    u0[...] = _dot(uf[...], wu[...])
    b0[...] = _dot(bf[...], wb[...])


def _sweep_kernel(abu, auu, aub, abb, u0r, b0r,
                  u1o, b1o, suo, sbo, u2p, b2p, u1s, b1s):
    i = pl.program_id(0)
    t = pl.program_id(1)
    j = jax.lax.rem(i + 1 + t, NB)
    row = pl.ds(i * BM, BM)
    col = pl.ds(j * BM, BM)

    @pl.when(t == 0)
    def _():
        u1s[row, :] = jnp.zeros((BM, D), jnp.float32)
        b1s[row, :] = jnp.zeros((BM, D), jnp.float32)
        u2p[...] = jnp.zeros_like(u2p)
        b2p[...] = jnp.zeros_like(b2p)

    u0j = u0r[col, :]
    b0j = b0r[col, :]
    u1s[row, :] += _dot(abu[...], b0j) + _dot(auu[...], u0j)
    b1s[row, :] += _dot(aub[...], u0j) + _dot(abb[...], b0j)

    @pl.when(j < i)
    def _():
        u1j = u1s[col, :]
        b1j = b1s[col, :]
        u2p[...] += _dot(abu[...], b1j) + _dot(auu[...], u1j)
        b2p[...] += _dot(aub[...], u1j) + _dot(abb[...], b1j)

    @pl.when(t == NB - 1)
    def _():
        u1i = u1s[row, :]
        b1i = b1s[row, :]
        u2p[...] += _dot(abu[...], b1i) + _dot(auu[...], u1i)
        b2p[...] += _dot(aub[...], u1i) + _dot(abb[...], b1i)
        u1o[...] = u1i
        b1o[...] = b1i
        suo[...] = u0r[row, :] + u1i
        sbo[...] = b0r[row, :] + b1i


def _upper_kernel(abu, auu, aub, abb, u1r, b1r, su, sb, u2p, b2p,
                  uh, bh, accu, accb):
    i = pl.program_id(0)
    t = pl.program_id(1)

    @pl.when(t == 0)
    def _():
        accu[...] = u2p[...]
        accb[...] = b2p[...]

    @pl.when(t > i)
    def _():
        col = pl.ds(t * BM, BM)
        u1j = u1r[col, :]
        b1j = b1r[col, :]
        accu[...] += _dot(abu[...], b1j) + _dot(auu[...], u1j)
        accb[...] += _dot(aub[...], u1j) + _dot(abb[...], b1j)

    @pl.when(t == NB - 1)
    def _():
        emb_u = (su[...] + accu[...]) * (1.0 / 3.0)
        emb_b = (sb[...] + accb[...]) * (1.0 / 3.0)
        nu = jnp.sqrt(jnp.sum(emb_u * emb_u, axis=-1, keepdims=True))
        nb = jnp.sqrt(jnp.sum(emb_b * emb_b, axis=-1, keepdims=True))
        uh[...] = emb_u / jnp.maximum(nu, EPS)
        bh[...] = emb_b / jnp.maximum(nb, EPS)


def _rot_spec():
    return pl.BlockSpec((BM, BM), lambda i, t: (i, jax.lax.rem(i + 1 + t, NB)))


def _upper_spec():
    return pl.BlockSpec(
        (BM, BM),
        lambda i, t: (i, jnp.minimum(jnp.maximum(t, i + 1), NB - 1)),
    )


def _resident_spec():
    return pl.BlockSpec((N, D), lambda i, t: (0, 0))


def _row_spec():
    return pl.BlockSpec((BM, D), lambda i, t: (i, 0))


def kernel(user_feat, biz_feat, adj_ub, adj_bu, adj_uu, adj_bb, W_user, W_biz):
    emb = jax.ShapeDtypeStruct((N, D), jnp.float32)

    u0, b0 = pl.pallas_call(
        _proj_kernel,
        grid=(NB,),
        in_specs=[
            pl.BlockSpec((BM, IN_DIM), lambda i: (i, 0)),
            pl.BlockSpec((BM, IN_DIM), lambda i: (i, 0)),
            pl.BlockSpec((IN_DIM, D), lambda i: (0, 0)),
            pl.BlockSpec((IN_DIM, D), lambda i: (0, 0)),
        ],
        out_specs=[
            pl.BlockSpec((BM, D), lambda i: (i, 0)),
            pl.BlockSpec((BM, D), lambda i: (i, 0)),
        ],
        out_shape=[emb, emb],
        compiler_params=pltpu.CompilerParams(
            dimension_semantics=("parallel",),
        ),
    )(user_feat, biz_feat, W_user, W_biz)

    u1, b1, su, sb, u2p, b2p = pl.pallas_call(
        _sweep_kernel,
        grid=(NB, NB),
        in_specs=[
            _rot_spec(), _rot_spec(), _rot_spec(), _rot_spec(),
            _resident_spec(), _resident_spec(),
        ],
        out_specs=[_row_spec()] * 6,
        out_shape=[emb] * 6,
        scratch_shapes=[
            pltpu.VMEM((N, D), jnp.float32),
            pltpu.VMEM((N, D), jnp.float32),
        ],
        compiler_params=pltpu.CompilerParams(
            dimension_semantics=("arbitrary", "arbitrary"),
        ),
    )(adj_bu, adj_uu, adj_ub, adj_bb, u0, b0)

    user_h, biz_h = pl.pallas_call(
        _upper_kernel,
        grid=(NB, NB),
        in_specs=[
            _upper_spec(), _upper_spec(), _upper_spec(), _upper_spec(),
            _resident_spec(), _resident_spec(),
            _row_spec(), _row_spec(), _row_spec(), _row_spec(),
        ],
        out_specs=[_row_spec(), _row_spec()],
        out_shape=[emb, emb],
        scratch_shapes=[
            pltpu.VMEM((BM, D), jnp.float32),
            pltpu.VMEM((BM, D), jnp.float32),
        ],
        compiler_params=pltpu.CompilerParams(
            dimension_semantics=("parallel", "arbitrary"),
        ),
    )(adj_bu, adj_uu, adj_ub, adj_bb, u1, b1, su, sb, u2p, b2p)

    return (user_h, biz_h)
